# Initial kernel scaffold; baseline (speedup 1.0000x reference)
import jax
import jax.numpy as jnp
from jax.experimental import pallas as pl


def kernel(f0, phone_label, phone_duration, midi_label, unvoiced_flag, W_f0, b_f0, phone_table, midi_table, W_unv, b_unv):
    raise NotImplementedError("write your pallas kernel here")


# speedup vs baseline: 1.8363x; 1.8363x over previous
import functools

import jax
import jax.numpy as jnp
from jax import lax
from jax.experimental import pallas as pl
from jax.experimental.pallas import tpu as pltpu
from jax.experimental.pallas import tpu_sc as plsc


def _gather_sc(phone_table, midi_table, pidx, midx, n_tokens):
    p_dim = phone_table.shape[1]
    m_dim = midi_table.shape[1]
    info = plsc.get_sparse_core_info()
    nw = info.num_cores * info.num_subcores
    n_w = n_tokens // nw
    ch = 256
    n_ch = n_w // ch
    mesh = plsc.VectorSubcoreMesh(core_axis_name="c", subcore_axis_name="s")

    @functools.partial(
        pl.kernel,
        mesh=mesh,
        compiler_params=pltpu.CompilerParams(use_tc_tiling_on_sc=False),
        out_type=[
            jax.ShapeDtypeStruct((n_tokens, p_dim), jnp.float32),
            jax.ShapeDtypeStruct((n_tokens, m_dim), jnp.float32),
        ],
        scratch_types=[
            pltpu.VMEM((n_w,), jnp.int32),
            pltpu.VMEM((n_w,), jnp.int32),
            pltpu.VMEM((ch, p_dim), jnp.float32),
            pltpu.VMEM((ch, m_dim), jnp.float32),
            pltpu.SemaphoreType.DMA,
        ],
    )
    def k(ptab, mtab, pidx_h, midx_h, pout, mout, pidx_v, midx_v, prows, mrows, sem):
        wid = lax.axis_index("s") * info.num_cores + lax.axis_index("c")
        base = wid * n_w
        pltpu.sync_copy(pidx_h.at[pl.ds(base, n_w)], pidx_v)
        pltpu.sync_copy(midx_h.at[pl.ds(base, n_w)], midx_v)
        for i in range(n_ch):
            off = i * ch
            pltpu.async_copy(ptab.at[pidx_v.at[pl.ds(off, ch)]], prows, sem).wait()
            pltpu.sync_copy(prows, pout.at[pl.ds(base + off, ch)])
            pltpu.async_copy(mtab.at[midx_v.at[pl.ds(off, ch)]], mrows, sem).wait()
            pltpu.sync_copy(mrows, mout.at[pl.ds(base + off, ch)])

    return k(phone_table, midi_table, pidx, midx)


def _proj_tc(f0_flat, unv_flat, wf_row, bf_row, wu_row, bu_row, n_tokens):
    f0_dim = wf_row.shape[1]
    unv_dim = wu_row.shape[1]
    blk = 2048
    grid = (n_tokens // blk,)

    def body(f0_ref, unv_ref, wf_ref, bf_ref, wu_ref, bu_ref, fo_ref, uo_ref):
        fo_ref[...] = f0_ref[...] * wf_ref[...] + bf_ref[...]
        uo_ref[...] = unv_ref[...] * wu_ref[...] + bu_ref[...]

    return pl.pallas_call(
        body,
        grid=grid,
        in_specs=[
            pl.BlockSpec((blk, 1), lambda i: (i, 0)),
            pl.BlockSpec((blk, 1), lambda i: (i, 0)),
            pl.BlockSpec((1, f0_dim), lambda i: (0, 0)),
            pl.BlockSpec((1, f0_dim), lambda i: (0, 0)),
            pl.BlockSpec((1, unv_dim), lambda i: (0, 0)),
            pl.BlockSpec((1, unv_dim), lambda i: (0, 0)),
        ],
        out_specs=[
            pl.BlockSpec((blk, f0_dim), lambda i: (i, 0)),
            pl.BlockSpec((blk, unv_dim), lambda i: (i, 0)),
        ],
        out_shape=[
            jax.ShapeDtypeStruct((n_tokens, f0_dim), jnp.float32),
            jax.ShapeDtypeStruct((n_tokens, unv_dim), jnp.float32),
        ],
    )(f0_flat, unv_flat, wf_row, bf_row, wu_row, bu_row)


def kernel(f0, phone_label, phone_duration, midi_label, unvoiced_flag,
           W_f0, b_f0, phone_table, midi_table, W_unv, b_unv):
    b, s = phone_label.shape
    n = b * s
    f0_dim = W_f0.shape[0]
    unv_dim = W_unv.shape[0]
    p_dim = phone_table.shape[1]
    m_dim = midi_table.shape[1]

    pidx = phone_label.astype(jnp.int32).reshape(n)
    midx = midi_label.astype(jnp.int32).reshape(n)
    pout, mout = _gather_sc(phone_table, midi_table, pidx, midx, n)

    fo, uo = _proj_tc(
        f0.reshape(n, 1), unvoiced_flag.reshape(n, 1),
        W_f0.reshape(1, f0_dim), b_f0.reshape(1, f0_dim),
        W_unv.reshape(1, unv_dim), b_unv.reshape(1, unv_dim),
        n,
    )
    return (
        fo.reshape(b, s, f0_dim),
        pout.reshape(b, s, p_dim),
        mout.reshape(b, s, m_dim),
        uo.reshape(b, s, unv_dim),
    )


# speedup vs baseline: 1.8365x; 1.0001x over previous
import functools

import jax
import jax.numpy as jnp
from jax import lax
from jax.experimental import pallas as pl
from jax.experimental.pallas import tpu as pltpu
from jax.experimental.pallas import tpu_sc as plsc


def _gather_sc(phone_table, midi_table, pidx, midx, n_tokens):
    p_dim = phone_table.shape[1]
    m_dim = midi_table.shape[1]
    info = plsc.get_sparse_core_info()
    nw = info.num_cores * info.num_subcores
    n_w = n_tokens // nw
    ch = 256
    n_ch = n_w // ch
    mesh = plsc.VectorSubcoreMesh(core_axis_name="c", subcore_axis_name="s")

    @functools.partial(
        pl.kernel,
        mesh=mesh,
        compiler_params=pltpu.CompilerParams(use_tc_tiling_on_sc=False),
        out_type=[
            jax.ShapeDtypeStruct((n_tokens, p_dim), jnp.float32),
            jax.ShapeDtypeStruct((n_tokens, m_dim), jnp.float32),
        ],
        scratch_types=[
            pltpu.VMEM((n_w,), jnp.int32),
            pltpu.VMEM((n_w,), jnp.int32),
            pltpu.VMEM((2, ch, p_dim), jnp.float32),
            pltpu.VMEM((2, ch, m_dim), jnp.float32),
            pltpu.SemaphoreType.DMA,
            pltpu.SemaphoreType.DMA,
            pltpu.SemaphoreType.DMA,
            pltpu.SemaphoreType.DMA,
        ],
    )
    def k(ptab, mtab, pidx_h, midx_h, pout, mout, pidx_v, midx_v, prows, mrows,
          sem_gp, sem_gm, sem_op, sem_om):
        wid = lax.axis_index("s") * info.num_cores + lax.axis_index("c")
        base = wid * n_w
        pltpu.sync_copy(pidx_h.at[pl.ds(base, n_w)], pidx_v)
        pltpu.sync_copy(midx_h.at[pl.ds(base, n_w)], midx_v)

        def gathers(i, buf):
            off = i * ch
            gp = pltpu.async_copy(
                ptab.at[pidx_v.at[pl.ds(off, ch)]], prows.at[buf], sem_gp)
            gm = pltpu.async_copy(
                mtab.at[midx_v.at[pl.ds(off, ch)]], mrows.at[buf], sem_gm)
            return gp, gm

        g = {0: gathers(0, 0)}
        o = {}
        for i in range(n_ch):
            b = i & 1
            if i + 1 < n_ch:
                if i >= 1:
                    o[i - 1][0].wait()
                    o[i - 1][1].wait()
                g[i + 1] = gathers(i + 1, (i + 1) & 1)
            g[i][0].wait()
            g[i][1].wait()
            off = base + i * ch
            o[i] = (
                pltpu.async_copy(prows.at[b], pout.at[pl.ds(off, ch)], sem_op),
                pltpu.async_copy(mrows.at[b], mout.at[pl.ds(off, ch)], sem_om),
            )
        for i in (n_ch - 2, n_ch - 1):
            o[i][0].wait()
            o[i][1].wait()

    return k(phone_table, midi_table, pidx, midx)


def _proj_tc(f0_flat, unv_flat, wf_row, bf_row, wu_row, bu_row, n_tokens):
    f0_dim = wf_row.shape[1]
    unv_dim = wu_row.shape[1]
    blk = 2048
    grid = (n_tokens // blk,)

    def body(f0_ref, unv_ref, wf_ref, bf_ref, wu_ref, bu_ref, fo_ref, uo_ref):
        fo_ref[...] = f0_ref[...] * wf_ref[...] + bf_ref[...]
        uo_ref[...] = unv_ref[...] * wu_ref[...] + bu_ref[...]

    return pl.pallas_call(
        body,
        grid=grid,
        in_specs=[
            pl.BlockSpec((blk, 1), lambda i: (i, 0)),
            pl.BlockSpec((blk, 1), lambda i: (i, 0)),
            pl.BlockSpec((1, f0_dim), lambda i: (0, 0)),
            pl.BlockSpec((1, f0_dim), lambda i: (0, 0)),
            pl.BlockSpec((1, unv_dim), lambda i: (0, 0)),
            pl.BlockSpec((1, unv_dim), lambda i: (0, 0)),
        ],
        out_specs=[
            pl.BlockSpec((blk, f0_dim), lambda i: (i, 0)),
            pl.BlockSpec((blk, unv_dim), lambda i: (i, 0)),
        ],
        out_shape=[
            jax.ShapeDtypeStruct((n_tokens, f0_dim), jnp.float32),
            jax.ShapeDtypeStruct((n_tokens, unv_dim), jnp.float32),
        ],
    )(f0_flat, unv_flat, wf_row, bf_row, wu_row, bu_row)


def kernel(f0, phone_label, phone_duration, midi_label, unvoiced_flag,
           W_f0, b_f0, phone_table, midi_table, W_unv, b_unv):
    b, s = phone_label.shape
    n = b * s
    f0_dim = W_f0.shape[0]
    unv_dim = W_unv.shape[0]
    p_dim = phone_table.shape[1]
    m_dim = midi_table.shape[1]

    pidx = phone_label.astype(jnp.int32).reshape(n)
    midx = midi_label.astype(jnp.int32).reshape(n)
    pout, mout = _gather_sc(phone_table, midi_table, pidx, midx, n)

    fo, uo = _proj_tc(
        f0.reshape(n, 1), unvoiced_flag.reshape(n, 1),
        W_f0.reshape(1, f0_dim), b_f0.reshape(1, f0_dim),
        W_unv.reshape(1, unv_dim), b_unv.reshape(1, unv_dim),
        n,
    )
    return (
        fo.reshape(b, s, f0_dim),
        pout.reshape(b, s, p_dim),
        mout.reshape(b, s, m_dim),
        uo.reshape(b, s, unv_dim),
    )


# speedup vs baseline: 2.1971x; 1.1964x over previous
import functools

import jax
import jax.numpy as jnp
from jax import lax
from jax.experimental import pallas as pl
from jax.experimental.pallas import tpu as pltpu
from jax.experimental.pallas import tpu_sc as plsc


def _gather_sc(ptab_flat, mtab_flat, pidx, midx, n_tokens, p_dim, m_dim):
    p_words = ptab_flat.shape[0]
    m_words = mtab_flat.shape[0]
    info = plsc.get_sparse_core_info()
    nw = info.num_cores * info.num_subcores
    n_w = n_tokens // nw
    ch = 256
    n_ch = n_w // ch
    mesh = plsc.VectorSubcoreMesh(core_axis_name="c", subcore_axis_name="s")

    @functools.partial(
        pl.kernel,
        mesh=mesh,
        compiler_params=pltpu.CompilerParams(use_tc_tiling_on_sc=False),
        out_type=[
            jax.ShapeDtypeStruct((n_tokens * p_dim,), jnp.float32),
            jax.ShapeDtypeStruct((n_tokens * m_dim,), jnp.float32),
        ],
        scratch_types=[
            pltpu.VMEM((p_words,), jnp.float32),
            pltpu.VMEM((m_words,), jnp.float32),
            pltpu.VMEM((n_w,), jnp.int32),
            pltpu.VMEM((n_w,), jnp.int32),
            pltpu.VMEM((2 * ch * p_dim,), jnp.float32),
            pltpu.VMEM((2 * ch * m_dim,), jnp.float32),
            pltpu.SemaphoreType.DMA,
            pltpu.SemaphoreType.DMA,
            pltpu.SemaphoreType.DMA,
        ],
    )
    def k(ptab_h, mtab_h, pidx_h, midx_h, pout, mout,
          ptab_v, mtab_v, pidx_v, midx_v, pstage, mstage,
          sem_t, sem_op, sem_om):
        wid = lax.axis_index("s") * info.num_cores + lax.axis_index("c")
        base = wid * n_w
        ct = pltpu.async_copy(ptab_h, ptab_v, sem_t)
        cm = pltpu.async_copy(mtab_h, mtab_v, sem_t)
        ci = pltpu.async_copy(pidx_h.at[pl.ds(base, n_w)], pidx_v, sem_t)
        cj = pltpu.async_copy(midx_h.at[pl.ds(base, n_w)], midx_v, sem_t)
        ct.wait()
        cm.wait()
        ci.wait()
        cj.wait()

        o = {}
        for c in range(n_ch):
            b = c & 1
            if c >= 2:
                o[c - 2][0].wait()
                o[c - 2][1].wait()
            pbase = b * ch * p_dim
            mbase = b * ch * m_dim

            def body(g, _, c=c, pbase=pbase, mbase=mbase):
                pidxv = pidx_v[pl.ds(c * ch + g * 16, 16)]
                midxv = midx_v[pl.ds(c * ch + g * 16, 16)]
                for l in range(16):
                    pi = pidxv[l] * p_dim
                    sb = pbase + g * (16 * p_dim) + l * p_dim
                    for kk in range(p_dim // 16):
                        pstage[pl.ds(sb + kk * 16, 16)] = (
                            ptab_v[pl.ds(pi + kk * 16, 16)])
                    mi = midxv[l] * m_dim
                    sm = mbase + g * (16 * m_dim) + l * m_dim
                    for kk in range(m_dim // 16):
                        mstage[pl.ds(sm + kk * 16, 16)] = (
                            mtab_v[pl.ds(mi + kk * 16, 16)])
                return 0

            lax.fori_loop(0, ch // 16, body, 0)
            off = base + c * ch
            o[c] = (
                pltpu.async_copy(pstage.at[pl.ds(pbase, ch * p_dim)],
                                 pout.at[pl.ds(off * p_dim, ch * p_dim)], sem_op),
                pltpu.async_copy(mstage.at[pl.ds(mbase, ch * m_dim)],
                                 mout.at[pl.ds(off * m_dim, ch * m_dim)], sem_om),
            )
        for c in (n_ch - 2, n_ch - 1):
            o[c][0].wait()
            o[c][1].wait()

    return k(ptab_flat, mtab_flat, pidx, midx)


def _proj_tc(f0_flat, unv_flat, wf_row, bf_row, wu_row, bu_row, n_tokens):
    f0_dim = wf_row.shape[1]
    unv_dim = wu_row.shape[1]
    blk = 2048
    grid = (n_tokens // blk,)

    def body(f0_ref, unv_ref, wf_ref, bf_ref, wu_ref, bu_ref, fo_ref, uo_ref):
        fo_ref[...] = f0_ref[...] * wf_ref[...] + bf_ref[...]
        uo_ref[...] = unv_ref[...] * wu_ref[...] + bu_ref[...]

    return pl.pallas_call(
        body,
        grid=grid,
        in_specs=[
            pl.BlockSpec((blk, 1), lambda i: (i, 0)),
            pl.BlockSpec((blk, 1), lambda i: (i, 0)),
            pl.BlockSpec((1, f0_dim), lambda i: (0, 0)),
            pl.BlockSpec((1, f0_dim), lambda i: (0, 0)),
            pl.BlockSpec((1, unv_dim), lambda i: (0, 0)),
            pl.BlockSpec((1, unv_dim), lambda i: (0, 0)),
        ],
        out_specs=[
            pl.BlockSpec((blk, f0_dim), lambda i: (i, 0)),
            pl.BlockSpec((blk, unv_dim), lambda i: (i, 0)),
        ],
        out_shape=[
            jax.ShapeDtypeStruct((n_tokens, f0_dim), jnp.float32),
            jax.ShapeDtypeStruct((n_tokens, unv_dim), jnp.float32),
        ],
    )(f0_flat, unv_flat, wf_row, bf_row, wu_row, bu_row)


def kernel(f0, phone_label, phone_duration, midi_label, unvoiced_flag,
           W_f0, b_f0, phone_table, midi_table, W_unv, b_unv):
    b, s = phone_label.shape
    n = b * s
    f0_dim = W_f0.shape[0]
    unv_dim = W_unv.shape[0]
    p_dim = phone_table.shape[1]
    m_dim = midi_table.shape[1]

    pidx = phone_label.astype(jnp.int32).reshape(n)
    midx = midi_label.astype(jnp.int32).reshape(n)
    pout, mout = _gather_sc(phone_table.reshape(-1), midi_table.reshape(-1),
                            pidx, midx, n, p_dim, m_dim)

    fo, uo = _proj_tc(
        f0.reshape(n, 1), unvoiced_flag.reshape(n, 1),
        W_f0.reshape(1, f0_dim), b_f0.reshape(1, f0_dim),
        W_unv.reshape(1, unv_dim), b_unv.reshape(1, unv_dim),
        n,
    )
    return (
        fo.reshape(b, s, f0_dim),
        pout.reshape(b, s, p_dim),
        mout.reshape(b, s, m_dim),
        uo.reshape(b, s, unv_dim),
    )


# speedup vs baseline: 5.1659x; 2.3512x over previous
import functools

import jax
import jax.numpy as jnp
from jax import lax
from jax.experimental import pallas as pl
from jax.experimental.pallas import tpu as pltpu
from jax.experimental.pallas import tpu_sc as plsc


def _gather_sc(ptab_flat, mtab_t_flat, pidx, midx, n_tokens, p_dim, m_dim,
               m_vocab, n_b, s_len):
    p_words = ptab_flat.shape[0]
    m_words = mtab_t_flat.shape[0]
    info = plsc.get_sparse_core_info()
    nw = info.num_cores * info.num_subcores
    n_w = n_tokens // nw
    ch = 256
    n_ch = n_w // ch
    mesh = plsc.VectorSubcoreMesh(core_axis_name="c", subcore_axis_name="s")

    @functools.partial(
        pl.kernel,
        mesh=mesh,
        compiler_params=pltpu.CompilerParams(use_tc_tiling_on_sc=True,
                                             needs_layout_passes=False),
        out_type=[
            jax.ShapeDtypeStruct((n_tokens * p_dim,), jnp.float32),
            jax.ShapeDtypeStruct((n_b, m_dim, s_len), jnp.float32),
        ],
        scratch_types=[
            pltpu.VMEM((p_words,), jnp.float32),
            pltpu.VMEM((m_words,), jnp.float32),
            pltpu.VMEM((n_w,), jnp.int32),
            pltpu.VMEM((n_w,), jnp.int32),
            pltpu.VMEM((2 * ch * p_dim,), jnp.float32),
            pltpu.VMEM((2, m_dim, ch), jnp.float32),
            pltpu.SemaphoreType.DMA,
            pltpu.SemaphoreType.DMA,
            pltpu.SemaphoreType.DMA,
        ],
    )
    def k(ptab_h, mtab_h, pidx_h, midx_h, pout, mout,
          ptab_v, mtab_v, pidx_v, midx_v, pstage, mstage,
          sem_t, sem_op, sem_om):
        wid = lax.axis_index("s") * info.num_cores + lax.axis_index("c")
        base = wid * n_w
        bb = base // s_len
        s0 = base % s_len
        ct = pltpu.async_copy(ptab_h, ptab_v, sem_t)
        cm = pltpu.async_copy(mtab_h, mtab_v, sem_t)
        ci = pltpu.async_copy(pidx_h.at[pl.ds(base, n_w)], pidx_v, sem_t)
        cj = pltpu.async_copy(midx_h.at[pl.ds(base, n_w)], midx_v, sem_t)
        ct.wait()
        cm.wait()
        ci.wait()
        cj.wait()

        o = {}
        for c in range(n_ch):
            b = c & 1
            if c >= 2:
                o[c - 2][0].wait()
                o[c - 2][1].wait()

            pbase = b * ch * p_dim

            @plsc.parallel_loop(0, ch // 16)
            def _body(g, c=c, b=b, pbase=pbase):
                pidxv = pidx_v[pl.ds(c * ch + g * 16, 16)] * p_dim
                midxv = midx_v[pl.ds(c * ch + g * 16, 16)]
                for l in range(0, 16, 2):
                    loads = []
                    for t in (l, l + 1):
                        pi = pidxv[t]
                        loads.append([ptab_v[pl.ds(pi + kk * 16, 16)]
                                      for kk in range(p_dim // 16)])
                    for t in (l, l + 1):
                        sb = pbase + (g * 16 + t) * p_dim
                        for kk in range(p_dim // 16):
                            pstage[pl.ds(sb + kk * 16, 16)] = loads[t - l][kk]
                for d0 in range(0, m_dim, 8):
                    vals = [plsc.load_gather(mtab_v, [midxv + d * m_vocab])
                            for d in range(d0, d0 + 8)]
                    for i, d in enumerate(range(d0, d0 + 8)):
                        mstage[b, d, pl.ds(g * 16, 16)] = vals[i]

            off = base + c * ch
            o[c] = (
                pltpu.async_copy(pstage.at[pl.ds(pbase, ch * p_dim)],
                                 pout.at[pl.ds(off * p_dim, ch * p_dim)],
                                 sem_op),
                pltpu.async_copy(mstage.at[b],
                                 mout.at[bb, :, pl.ds(s0 + c * ch, ch)],
                                 sem_om),
            )
        for c in (n_ch - 2, n_ch - 1):
            o[c][0].wait()
            o[c][1].wait()

    return k(ptab_flat, mtab_t_flat, pidx, midx)


def _proj_tc(f0_bs, unv_bs, wf_col, bf_col, wu_col, bu_col):
    f0_dim = wf_col.shape[0]
    unv_dim = wu_col.shape[0]
    n_b, s_len = f0_bs.shape
    sblk = 256
    grid = (s_len // sblk,)

    def body(f0_ref, unv_ref, wf_ref, bf_ref, wu_ref, bu_ref, fo_ref, uo_ref):
        f0r = f0_ref[...]
        unr = unv_ref[...]
        fo_ref[...] = (wf_ref[...][None, :, :] * f0r[:, None, :]
                       + bf_ref[...][None, :, :])
        uo_ref[...] = (wu_ref[...][None, :, :] * unr[:, None, :]
                       + bu_ref[...][None, :, :])

    return pl.pallas_call(
        body,
        grid=grid,
        in_specs=[
            pl.BlockSpec((n_b, sblk), lambda j: (0, j)),
            pl.BlockSpec((n_b, sblk), lambda j: (0, j)),
            pl.BlockSpec((f0_dim, 1), lambda j: (0, 0)),
            pl.BlockSpec((f0_dim, 1), lambda j: (0, 0)),
            pl.BlockSpec((unv_dim, 1), lambda j: (0, 0)),
            pl.BlockSpec((unv_dim, 1), lambda j: (0, 0)),
        ],
        out_specs=[
            pl.BlockSpec((n_b, f0_dim, sblk), lambda j: (0, 0, j)),
            pl.BlockSpec((n_b, unv_dim, sblk), lambda j: (0, 0, j)),
        ],
        out_shape=[
            jax.ShapeDtypeStruct((n_b, f0_dim, s_len), jnp.float32),
            jax.ShapeDtypeStruct((n_b, unv_dim, s_len), jnp.float32),
        ],
    )(f0_bs, unv_bs, wf_col, bf_col, wu_col, bu_col)


def kernel(f0, phone_label, phone_duration, midi_label, unvoiced_flag,
           W_f0, b_f0, phone_table, midi_table, W_unv, b_unv):
    b, s = phone_label.shape
    n = b * s
    f0_dim = W_f0.shape[0]
    unv_dim = W_unv.shape[0]
    p_dim = phone_table.shape[1]
    m_dim = midi_table.shape[1]
    m_vocab = midi_table.shape[0]

    pidx = phone_label.astype(jnp.int32).reshape(n)
    midx = midi_label.astype(jnp.int32).reshape(n)
    pout, mout3 = _gather_sc(
        phone_table.reshape(-1), midi_table.T.reshape(-1),
        pidx, midx, n, p_dim, m_dim, m_vocab, b, s)

    fo3, uo3 = _proj_tc(
        f0.reshape(b, s), unvoiced_flag.reshape(b, s),
        W_f0, b_f0.reshape(f0_dim, 1),
        W_unv, b_unv.reshape(unv_dim, 1),
    )
    return (
        jnp.swapaxes(fo3, 1, 2),
        pout.reshape(b, s, p_dim),
        jnp.swapaxes(mout3, 1, 2),
        jnp.swapaxes(uo3, 1, 2),
    )


# speedup vs baseline: 5.2244x; 1.0113x over previous
import functools

import jax
import jax.numpy as jnp
from jax import lax
from jax.experimental import pallas as pl
from jax.experimental.pallas import tpu as pltpu
from jax.experimental.pallas import tpu_sc as plsc


def _gather_sc(ptab_flat, mtab_t, pidx, midx, n_tokens, p_dim, m_dim,
               m_vocab, n_b, s_len):
    p_words = ptab_flat.shape[0]
    info = plsc.get_sparse_core_info()
    nw = info.num_cores * info.num_subcores
    n_w = n_tokens // nw
    ch = 256
    n_ch = n_w // ch
    mesh = plsc.VectorSubcoreMesh(core_axis_name="c", subcore_axis_name="s")

    @functools.partial(
        pl.kernel,
        mesh=mesh,
        compiler_params=pltpu.CompilerParams(use_tc_tiling_on_sc=True,
                                             needs_layout_passes=False),
        out_type=[
            jax.ShapeDtypeStruct((n_tokens * p_dim,), jnp.float32),
            jax.ShapeDtypeStruct((n_b, m_dim, s_len), jnp.float32),
        ],
        scratch_types=[
            pltpu.VMEM((p_words,), jnp.float32),
            pltpu.VMEM((m_dim, m_vocab), jnp.float32),
            pltpu.VMEM((n_w,), jnp.int32),
            pltpu.VMEM((n_w,), jnp.int32),
            pltpu.VMEM((2 * ch * p_dim,), jnp.float32),
            pltpu.VMEM((2, m_dim, ch), jnp.float32),
            pltpu.SemaphoreType.DMA,
            pltpu.SemaphoreType.DMA,
            pltpu.SemaphoreType.DMA,
        ],
    )
    def k(ptab_h, mtab_h, pidx_h, midx_h, pout, mout,
          ptab_v, mtab_v, pidx_v, midx_v, pstage, mstage,
          sem_t, sem_op, sem_om):
        wid = lax.axis_index("s") * info.num_cores + lax.axis_index("c")
        base = wid * n_w
        bb = base // s_len
        s0 = base % s_len
        ct = pltpu.async_copy(ptab_h, ptab_v, sem_t)
        cm = pltpu.async_copy(mtab_h, mtab_v, sem_t)
        ci = pltpu.async_copy(pidx_h.at[pl.ds(base, n_w)], pidx_v, sem_t)
        cj = pltpu.async_copy(midx_h.at[pl.ds(base, n_w)], midx_v, sem_t)
        ct.wait()
        cm.wait()
        ci.wait()
        cj.wait()

        o = {}
        for c in range(n_ch):
            b = c & 1
            if c >= 2:
                o[c - 2][0].wait()
                o[c - 2][1].wait()

            pbase = b * ch * p_dim

            @plsc.parallel_loop(0, ch // 16)
            def _body(g, c=c, b=b, pbase=pbase):
                pidxv = pidx_v[pl.ds(c * ch + g * 16, 16)] * p_dim
                midxv = midx_v[pl.ds(c * ch + g * 16, 16)]
                for l in range(0, 16, 2):
                    loads = []
                    for t in (l, l + 1):
                        pi = pidxv[t]
                        loads.append([ptab_v[pl.ds(pi + kk * 16, 16)]
                                      for kk in range(p_dim // 16)])
                    for t in (l, l + 1):
                        sb = pbase + (g * 16 + t) * p_dim
                        for kk in range(p_dim // 16):
                            pstage[pl.ds(sb + kk * 16, 16)] = loads[t - l][kk]
                for d0 in range(0, m_dim, 8):
                    vals = [plsc.load_gather(mtab_v.at[d], [midxv])
                            for d in range(d0, d0 + 8)]
                    for i, d in enumerate(range(d0, d0 + 8)):
                        mstage[b, d, pl.ds(g * 16, 16)] = vals[i]

            off = base + c * ch
            o[c] = (
                pltpu.async_copy(pstage.at[pl.ds(pbase, ch * p_dim)],
                                 pout.at[pl.ds(off * p_dim, ch * p_dim)],
                                 sem_op),
                pltpu.async_copy(mstage.at[b],
                                 mout.at[bb, :, pl.ds(s0 + c * ch, ch)],
                                 sem_om),
            )
        for c in (n_ch - 2, n_ch - 1):
            o[c][0].wait()
            o[c][1].wait()

    return k(ptab_flat, mtab_t, pidx, midx)


def _proj_tc(f0_bs, unv_bs, wf_col, bf_col, wu_col, bu_col):
    f0_dim = wf_col.shape[0]
    unv_dim = wu_col.shape[0]
    n_b, s_len = f0_bs.shape
    sblk = 256
    grid = (s_len // sblk,)

    def body(f0_ref, unv_ref, wf_ref, bf_ref, wu_ref, bu_ref, fo_ref, uo_ref):
        f0r = f0_ref[...]
        unr = unv_ref[...]
        fo_ref[...] = (wf_ref[...][None, :, :] * f0r[:, None, :]
                       + bf_ref[...][None, :, :])
        uo_ref[...] = (wu_ref[...][None, :, :] * unr[:, None, :]
                       + bu_ref[...][None, :, :])

    return pl.pallas_call(
        body,
        grid=grid,
        in_specs=[
            pl.BlockSpec((n_b, sblk), lambda j: (0, j)),
            pl.BlockSpec((n_b, sblk), lambda j: (0, j)),
            pl.BlockSpec((f0_dim, 1), lambda j: (0, 0)),
            pl.BlockSpec((f0_dim, 1), lambda j: (0, 0)),
            pl.BlockSpec((unv_dim, 1), lambda j: (0, 0)),
            pl.BlockSpec((unv_dim, 1), lambda j: (0, 0)),
        ],
        out_specs=[
            pl.BlockSpec((n_b, f0_dim, sblk), lambda j: (0, 0, j)),
            pl.BlockSpec((n_b, unv_dim, sblk), lambda j: (0, 0, j)),
        ],
        out_shape=[
            jax.ShapeDtypeStruct((n_b, f0_dim, s_len), jnp.float32),
            jax.ShapeDtypeStruct((n_b, unv_dim, s_len), jnp.float32),
        ],
    )(f0_bs, unv_bs, wf_col, bf_col, wu_col, bu_col)


def kernel(f0, phone_label, phone_duration, midi_label, unvoiced_flag,
           W_f0, b_f0, phone_table, midi_table, W_unv, b_unv):
    b, s = phone_label.shape
    n = b * s
    f0_dim = W_f0.shape[0]
    unv_dim = W_unv.shape[0]
    p_dim = phone_table.shape[1]
    m_dim = midi_table.shape[1]
    m_vocab = midi_table.shape[0]

    pidx = phone_label.astype(jnp.int32).reshape(n)
    midx = midi_label.astype(jnp.int32).reshape(n)
    pout, mout3 = _gather_sc(
        phone_table.reshape(-1), midi_table.T,
        pidx, midx, n, p_dim, m_dim, m_vocab, b, s)

    fo3, uo3 = _proj_tc(
        f0.reshape(b, s), unvoiced_flag.reshape(b, s),
        W_f0, b_f0.reshape(f0_dim, 1),
        W_unv, b_unv.reshape(unv_dim, 1),
    )
    return (
        jnp.swapaxes(fo3, 1, 2),
        pout.reshape(b, s, p_dim),
        jnp.swapaxes(mout3, 1, 2),
        jnp.swapaxes(uo3, 1, 2),
    )
